# Initial kernel scaffold; baseline (speedup 1.0000x reference)
import jax
import jax.numpy as jnp
from jax.experimental import pallas as pl


def kernel(edge_index, edge_attr, batch, node_emb, We, be, W1, b1, g1, bt1, W2, b2, g, b, Wh1, bh1, Wh2, bh2):
    raise NotImplementedError("write your pallas kernel here")


# speedup vs baseline: 3.2269x; 3.2269x over previous
import functools

import jax
import jax.numpy as jnp
from jax import lax
from jax.experimental import pallas as pl
from jax.experimental.pallas import tpu as pltpu
from jax.experimental.pallas import tpu_sc as plsc

C = 128
NTILES = 32


def _e_body_l0(attr_ref, w_ref, b_ref, x0_ref, o_ref):
    a = jnp.dot(attr_ref[...], w_ref[...], preferred_element_type=jnp.float32)
    o_ref[...] = a + b_ref[...] + x0_ref[...]


def _e_body(attr_ref, w_ref, b_ref, o_ref):
    a = jnp.dot(attr_ref[...], w_ref[...], preferred_element_type=jnp.float32)
    o_ref[...] = a + b_ref[...]


def _edge_transform(attr_pad, w, b, x0, layer0, eblk):
    e_pad = attr_pad.shape[0]
    grid = (e_pad // eblk,)
    in_specs = [
        pl.BlockSpec((eblk, attr_pad.shape[1]), lambda i: (i, 0)),
        pl.BlockSpec((attr_pad.shape[1], 128), lambda i: (0, 0)),
        pl.BlockSpec((1, 128), lambda i: (0, 0)),
    ]
    args = [attr_pad, w, b]
    if layer0:
        in_specs.append(pl.BlockSpec((1, 128), lambda i: (0, 0)))
        args.append(x0)
        body = _e_body_l0
    else:
        body = _e_body
    return pl.pallas_call(
        body,
        grid=grid,
        in_specs=in_specs,
        out_specs=pl.BlockSpec((eblk, 128), lambda i: (i, 0)),
        out_shape=jax.ShapeDtypeStruct((e_pad, 128), jnp.float32),
    )(*args)


def _make_sc_edge(n_pad, e_pad, layer0):
    ept = e_pad // NTILES
    nch = ept // C
    rows_pt = n_pad // 16
    wb = rows_pt // C

    mesh = plsc.VectorSubcoreMesh(core_axis_name="c", subcore_axis_name="s")

    scratch = [
        pltpu.VMEM((C,), jnp.int32),
        pltpu.VMEM((C,), jnp.int32),
        pltpu.VMEM((C, 128), jnp.float32),
        pltpu.VMEM((C, 128), jnp.float32),
        pltpu.VMEM_SHARED((n_pad, 128), jnp.float32),
        pltpu.SemaphoreType.DMA,
        pltpu.SemaphoreType.DMA,
        pltpu.SemaphoreType.DMA,
    ]

    def body(e_hbm, src_hbm, dst2_hbm, x_hbm, out_hbm,
             src_v, dst_v, buf_e, buf_x, aggr_sh, sem_i, sem_e, sem_x):
        cid = lax.axis_index("c")
        sid = lax.axis_index("s")
        wid = cid * 16 + sid
        base = wid * ept
        row0 = sid * rows_pt

        zero16 = jnp.zeros((16,), jnp.float32)

        def zrow(i, _):
            for j in range(8):
                buf_e[i, pl.ds(j * 16, 16)] = zero16
            return 0

        lax.fori_loop(0, C, zrow, 0)
        for r in range(wb):
            pltpu.sync_copy(buf_e, aggr_sh.at[pl.ds(row0 + r * C, C), :])
        plsc.subcore_barrier()

        def chunk(c, _):
            off = base + c * C
            cp_e = pltpu.async_copy(e_hbm.at[pl.ds(off, C), :], buf_e, sem_e)
            cp_d = pltpu.async_copy(dst2_hbm.at[wid * nch + c], dst_v, sem_i)
            if not layer0:
                pltpu.sync_copy(src_hbm.at[pl.ds(off, C)], src_v)
                cp_x = pltpu.async_copy(x_hbm.at[src_v], buf_x, sem_x)
            cp_e.wait()
            if not layer0:
                cp_x.wait()

                def row(i, _):
                    for j in range(8):
                        s = pl.ds(j * 16, 16)
                        buf_e[i, s] = jnp.maximum(buf_e[i, s] + buf_x[i, s], 0.0)
                    return 0
            else:
                def row(i, _):
                    for j in range(8):
                        s = pl.ds(j * 16, 16)
                        buf_e[i, s] = jnp.maximum(buf_e[i, s], 0.0)
                    return 0

            lax.fori_loop(0, C, row, 0)
            cp_d.wait()
            pltpu.sync_copy(buf_e, aggr_sh.at[dst_v], add=True)
            return 0

        lax.fori_loop(0, nch, chunk, 0)
        plsc.subcore_barrier()

        for r in range(wb):
            pltpu.sync_copy(aggr_sh.at[pl.ds(row0 + r * C, C), :], buf_e)
            pltpu.sync_copy(buf_e, out_hbm.at[cid, pl.ds(row0 + r * C, C), :])

    k = pl.kernel(
        body,
        out_type=jax.ShapeDtypeStruct((2, n_pad, 128), jnp.float32),
        mesh=mesh,
        scratch_types=scratch,
    )
    return k


def _node_body(x_ref, a0_ref, a1_ref, w1_ref, b1_ref, w2_ref, b2_ref, o_ref):
    h = x_ref[...] + a0_ref[...] + a1_ref[...]
    h = jnp.dot(h, w1_ref[...], preferred_element_type=jnp.float32) + b1_ref[...]
    h = jnp.maximum(h, 0.0)
    h = jnp.dot(h, w2_ref[...], preferred_element_type=jnp.float32) + b2_ref[...]
    o_ref[...] = jnp.maximum(h, 0.0)


def _node_mlp(x, a0, a1, w1, b1, w2, b2, n_pad, nblk):
    grid = (n_pad // nblk,)
    if x.shape[0] == 1:
        x_spec = pl.BlockSpec((1, 128), lambda i: (0, 0))
    else:
        x_spec = pl.BlockSpec((nblk, 128), lambda i: (i, 0))
    return pl.pallas_call(
        _node_body,
        grid=grid,
        in_specs=[
            x_spec,
            pl.BlockSpec((nblk, 128), lambda i: (i, 0)),
            pl.BlockSpec((nblk, 128), lambda i: (i, 0)),
            pl.BlockSpec((128, 128), lambda i: (0, 0)),
            pl.BlockSpec((1, 128), lambda i: (0, 0)),
            pl.BlockSpec((128, 128), lambda i: (0, 0)),
            pl.BlockSpec((1, 128), lambda i: (0, 0)),
        ],
        out_specs=pl.BlockSpec((nblk, 128), lambda i: (i, 0)),
        out_shape=jax.ShapeDtypeStruct((n_pad, 128), jnp.float32),
    )(x, a0, a1, w1, b1, w2, b2)


def _make_pool(n_graphs, n_pad, pblk):
    nb = n_pad // pblk

    def body(b3_ref, x_ref, wh1_ref, bh1_ref, wh2_ref, bh2_ref, o_ref,
             sums, cnts):
        i = pl.program_id(0)
        bb = b3_ref[0, 0, :]
        gid = lax.broadcasted_iota(jnp.int32, (n_graphs, pblk), 0)
        eq = jnp.where(gid == bb[None, :], 1.0, 0.0).astype(jnp.float32)
        ps = jnp.dot(eq, x_ref[...], preferred_element_type=jnp.float32)
        pc = jnp.sum(eq, axis=1, keepdims=True)

        @pl.when(i == 0)
        def _():
            sums[...] = ps
            cnts[...] = pc

        @pl.when(i > 0)
        def _():
            sums[...] += ps
            cnts[...] += pc

        @pl.when(i == nb - 1)
        def _():
            g = sums[...] / jnp.maximum(cnts[...], 1.0)
            h = jnp.dot(g, wh1_ref[...], preferred_element_type=jnp.float32)
            h = jnp.maximum(h + bh1_ref[...], 0.0)
            o_ref[...] = (
                jnp.dot(h, wh2_ref[...], preferred_element_type=jnp.float32)
                + bh2_ref[...])

    return pl.pallas_call(
        body,
        grid=(nb,),
        in_specs=[
            pl.BlockSpec((1, 1, pblk), lambda i: (i, 0, 0)),
            pl.BlockSpec((pblk, 128), lambda i: (i, 0)),
            pl.BlockSpec((128, 128), lambda i: (0, 0)),
            pl.BlockSpec((1, 128), lambda i: (0, 0)),
            pl.BlockSpec((128, 128), lambda i: (0, 0)),
            pl.BlockSpec((1, 128), lambda i: (0, 0)),
        ],
        out_specs=pl.BlockSpec((n_graphs, 128), lambda i: (0, 0)),
        out_shape=jax.ShapeDtypeStruct((n_graphs, 128), jnp.float32),
        scratch_shapes=[
            pltpu.VMEM((n_graphs, 128), jnp.float32),
            pltpu.VMEM((n_graphs, 1), jnp.float32),
        ],
    )


def kernel(edge_index, edge_attr, batch, node_emb, We, be, W1, b1, g1, bt1,
           W2, b2, g, b, Wh1, bh1, Wh2, bh2):
    n = batch.shape[0]
    e_cnt = edge_index.shape[1]
    n_layers = We.shape[0]
    n_graphs = 64
    bn_inv = 1.0 / jnp.sqrt(1.0 + 1e-5)

    e_pad = ((e_cnt + NTILES * C - 1) // (NTILES * C)) * (NTILES * C)
    n_pad = ((n + 1 + 16 * C - 1) // (16 * C)) * (16 * C)
    pblk = 1280
    nblk = 1280
    eblk = 2048

    src = edge_index[0]
    dst = edge_index[1]
    pad_e = e_pad - e_cnt
    src_p = jnp.concatenate([src, jnp.zeros((pad_e,), jnp.int32)]).astype(jnp.int32)
    dst_p = jnp.concatenate([dst, jnp.full((pad_e,), n, jnp.int32)]).astype(jnp.int32)
    dst2 = dst_p.reshape(e_pad // C, C)
    attr_p = jnp.pad(edge_attr, ((0, pad_e), (0, 1)))
    batch_p = jnp.concatenate(
        [batch.astype(jnp.int32), jnp.full((n_pad - n,), n_graphs, jnp.int32)])
    batch3 = batch_p.reshape(n_pad // pblk, 1, pblk)

    we_p = jnp.pad(We, ((0, 0), (0, 1), (0, 0)))
    s1 = bn_inv * g1
    w1f = W1 * s1[:, None, :]
    b1f = (b1 * s1 + bt1).reshape(n_layers, 1, 128)
    s2 = bn_inv * g
    w2f = W2 * s2[:, None, :]
    b2f = (b2 * s2 + b).reshape(n_layers, 1, 128)
    be2 = be.reshape(n_layers, 1, 128)
    wh2p = jnp.pad(Wh2, ((0, 0), (0, 128 - Wh2.shape[1])))
    bh2p = jnp.pad(bh2, (0, 128 - bh2.shape[0])).reshape(1, 128)
    bh1r = bh1.reshape(1, 128)

    sc_l0 = _make_sc_edge(n_pad, e_pad, layer0=True)
    sc_gen = _make_sc_edge(n_pad, e_pad, layer0=False)

    x = node_emb
    for l in range(n_layers):
        e = _edge_transform(attr_p, we_p[l], be2[l], node_emb, l == 0, eblk)
        if l == 0:
            parts = sc_l0(e, src_p, dst2, jnp.zeros((n_pad, 128), jnp.float32))
        else:
            parts = sc_gen(e, src_p, dst2, x)
        x = _node_mlp(x, parts[0], parts[1], w1f[l], b1f[l], w2f[l], b2f[l],
                      n_pad, nblk)

    out = _make_pool(n_graphs, n_pad, pblk)(
        batch3, x, Wh1, bh1r, wh2p, bh2p)
    return out[:, :Wh2.shape[1]]
